# baseline (device time: 6762 ns/iter reference)
import jax
import jax.numpy as jnp
from jax import lax
from jax.experimental import pallas as pl
from jax.experimental.pallas import tpu as pltpu


def kernel(x):
    m, n = x.shape

    def body(x_ref, out_ref, xv, send_buf, in_sem, own_sem, send_sem, recv_sem):
        my_x = lax.axis_index("x")
        my_y = lax.axis_index("y")
        my_z = lax.axis_index("z")
        peer = (1 - my_x, my_y, my_z)

        in_dma = pltpu.make_async_copy(x_ref, xv, in_sem)
        in_dma.start()

        barrier_sem = pltpu.get_barrier_semaphore()
        pl.semaphore_signal(
            barrier_sem, inc=1, device_id=peer,
            device_id_type=pl.DeviceIdType.MESH,
        )

        in_dma.wait()
        send_buf[...] = xv[...].astype(jnp.bfloat16)

        pl.semaphore_wait(barrier_sem, 1)

        rdma = pltpu.make_async_remote_copy(
            src_ref=send_buf,
            dst_ref=out_ref.at[pl.ds(my_x * m, m), :],
            send_sem=send_sem,
            recv_sem=recv_sem,
            device_id=peer,
            device_id_type=pl.DeviceIdType.MESH,
        )
        rdma.start()

        own_dma = pltpu.make_async_copy(
            send_buf, out_ref.at[pl.ds(my_x * m, m), :], own_sem
        )
        own_dma.start()

        recv = pltpu.make_async_remote_copy(
            src_ref=send_buf,
            dst_ref=out_ref.at[pl.ds((1 - my_x) * m, m), :],
            send_sem=send_sem,
            recv_sem=recv_sem,
            device_id=peer,
            device_id_type=pl.DeviceIdType.MESH,
        )
        own_dma.wait()
        recv.wait_recv()
        rdma.wait_send()

    return pl.pallas_call(
        body,
        out_shape=jax.ShapeDtypeStruct((2 * m, n), jnp.bfloat16),
        in_specs=[pl.BlockSpec(memory_space=pl.ANY)],
        out_specs=pl.BlockSpec(memory_space=pl.ANY),
        scratch_shapes=[
            pltpu.VMEM((m, n), x.dtype),
            pltpu.VMEM((m, n), jnp.bfloat16),
            pltpu.SemaphoreType.DMA,
            pltpu.SemaphoreType.DMA,
            pltpu.SemaphoreType.DMA,
            pltpu.SemaphoreType.DMA,
        ],
        compiler_params=pltpu.CompilerParams(collective_id=0),
    )(x)


# device time: 6697 ns/iter; 1.0097x vs baseline; 1.0097x over previous
import jax
import jax.numpy as jnp
from jax import lax
from jax.experimental import pallas as pl
from jax.experimental.pallas import tpu as pltpu

N_CHUNKS = 2


def kernel(x):
    m, n = x.shape
    h = m // N_CHUNKS

    def body(x_ref, out_ref, xv, send_buf, in_sems, own_sem, send_sems, recv_sems):
        my_x = lax.axis_index("x")
        my_y = lax.axis_index("y")
        my_z = lax.axis_index("z")
        peer = (1 - my_x, my_y, my_z)

        in_dmas = [
            pltpu.make_async_copy(
                x_ref.at[pl.ds(c * h, h), :], xv.at[pl.ds(c * h, h), :],
                in_sems.at[c],
            )
            for c in range(N_CHUNKS)
        ]
        for dma in in_dmas:
            dma.start()

        barrier_sem = pltpu.get_barrier_semaphore()
        pl.semaphore_signal(
            barrier_sem, inc=1, device_id=peer,
            device_id_type=pl.DeviceIdType.MESH,
        )

        in_dmas[0].wait()
        send_buf[pl.ds(0, h), :] = xv[pl.ds(0, h), :].astype(jnp.bfloat16)

        pl.semaphore_wait(barrier_sem, 1)

        rdmas = [
            pltpu.make_async_remote_copy(
                src_ref=send_buf.at[pl.ds(c * h, h), :],
                dst_ref=out_ref.at[pl.ds(my_x * m + c * h, h), :],
                send_sem=send_sems.at[c],
                recv_sem=recv_sems.at[c],
                device_id=peer,
                device_id_type=pl.DeviceIdType.MESH,
            )
            for c in range(N_CHUNKS)
        ]
        rdmas[0].start()

        in_dmas[1].wait()
        send_buf[pl.ds(h, h), :] = xv[pl.ds(h, h), :].astype(jnp.bfloat16)
        rdmas[1].start()

        own_dma = pltpu.make_async_copy(
            send_buf, out_ref.at[pl.ds(my_x * m, m), :], own_sem
        )
        own_dma.start()

        recvs = [
            pltpu.make_async_remote_copy(
                src_ref=send_buf.at[pl.ds(c * h, h), :],
                dst_ref=out_ref.at[pl.ds((1 - my_x) * m + c * h, h), :],
                send_sem=send_sems.at[c],
                recv_sem=recv_sems.at[c],
                device_id=peer,
                device_id_type=pl.DeviceIdType.MESH,
            )
            for c in range(N_CHUNKS)
        ]
        own_dma.wait()
        for c in range(N_CHUNKS):
            recvs[c].wait_recv()
        for c in range(N_CHUNKS):
            rdmas[c].wait_send()

    x = pltpu.with_memory_space_constraint(x, pltpu.MemorySpace.HBM)
    return pl.pallas_call(
        body,
        out_shape=jax.ShapeDtypeStruct((2 * m, n), jnp.bfloat16),
        in_specs=[pl.BlockSpec(memory_space=pltpu.MemorySpace.HBM)],
        out_specs=pl.BlockSpec(memory_space=pltpu.MemorySpace.HBM),
        scratch_shapes=[
            pltpu.VMEM((m, n), x.dtype),
            pltpu.VMEM((m, n), jnp.bfloat16),
            pltpu.SemaphoreType.DMA((N_CHUNKS,)),
            pltpu.SemaphoreType.DMA,
            pltpu.SemaphoreType.DMA((N_CHUNKS,)),
            pltpu.SemaphoreType.DMA((N_CHUNKS,)),
        ],
        compiler_params=pltpu.CompilerParams(collective_id=0),
    )(x)
